# baseline (speedup 1.0000x reference)
import functools

import jax
import jax.numpy as jnp
from jax.experimental import pallas as pl

G_SEG = 256
_SLOPE = 0.2


def _sig(x):
    return 1.0 / (1.0 + jnp.exp(-x))


def _lrelu(x):
    return jnp.where(x > 0, x, _SLOPE * x)


def _blk(n, target):
    b = min(n, target)
    while n % b or b % 8:
        b -= 1
    return b


def _dot(a, b):
    return jnp.dot(a, b, preferred_element_type=jnp.float32,
                   precision=jax.lax.Precision.HIGHEST)


def _dual_mm_kernel(x_ref, w1_ref, b1_ref, w2_ref, b2_ref, o1_ref, o2_ref, *, act2):
    x = x_ref[...]
    o1_ref[...] = _dot(x, w1_ref[...]) + b1_ref[...]
    y = _dot(x, w2_ref[...]) + b2_ref[...]
    o2_ref[...] = _lrelu(y) if act2 else y


def _dual_mm(x, W1, b1, W2, b2, act2):
    n, k = x.shape
    bn = _blk(n, 5000)
    h1 = W1.shape[1]
    h2 = W2.shape[1]
    return pl.pallas_call(
        functools.partial(_dual_mm_kernel, act2=act2),
        grid=(n // bn,),
        in_specs=[
            pl.BlockSpec((bn, k), lambda i: (i, 0)),
            pl.BlockSpec((k, h1), lambda i: (0, 0)),
            pl.BlockSpec((1, h1), lambda i: (0, 0)),
            pl.BlockSpec((k, h2), lambda i: (0, 0)),
            pl.BlockSpec((1, h2), lambda i: (0, 0)),
        ],
        out_specs=[
            pl.BlockSpec((bn, h1), lambda i: (i, 0)),
            pl.BlockSpec((bn, h2), lambda i: (i, 0)),
        ],
        out_shape=[
            jax.ShapeDtypeStruct((n, h1), jnp.float32),
            jax.ShapeDtypeStruct((n, h2), jnp.float32),
        ],
    )(x, W1, b1.reshape(1, -1), W2, b2.reshape(1, -1))


def _proj_attn_kernel(x_ref, w_ref, a_ref, hs_ref, ls_ref):
    hs = _dot(x_ref[...], w_ref[...])
    hs_ref[...] = hs
    ls_ref[...] = _dot(hs, a_ref[...])


def _proj_attn(x, W, a):
    n, k = x.shape
    h = W.shape[1]
    bn = _blk(n, 5000)
    return pl.pallas_call(
        _proj_attn_kernel,
        grid=(n // bn,),
        in_specs=[
            pl.BlockSpec((bn, k), lambda i: (i, 0)),
            pl.BlockSpec((k, h), lambda i: (0, 0)),
            pl.BlockSpec((h, 1), lambda i: (0, 0)),
        ],
        out_specs=[
            pl.BlockSpec((bn, h), lambda i: (i, 0)),
            pl.BlockSpec((bn, 1), lambda i: (i, 0)),
        ],
        out_shape=[
            jax.ShapeDtypeStruct((n, h), jnp.float32),
            jax.ShapeDtypeStruct((n, 1), jnp.float32),
        ],
    )(x, W, a.reshape(-1, 1))


def _attn_scalar_kernel(x_ref, w_ref, a_ref, t_ref):
    t_ref[...] = _dot(_dot(x_ref[...], w_ref[...]), a_ref[...])


def _attn_scalar(x, W, a):
    n, k = x.shape
    h = W.shape[1]
    bn = _blk(n, 5000)
    return pl.pallas_call(
        _attn_scalar_kernel,
        grid=(n // bn,),
        in_specs=[
            pl.BlockSpec((bn, k), lambda i: (i, 0)),
            pl.BlockSpec((k, h), lambda i: (0, 0)),
            pl.BlockSpec((h, 1), lambda i: (0, 0)),
        ],
        out_specs=pl.BlockSpec((bn, 1), lambda i: (i, 0)),
        out_shape=jax.ShapeDtypeStruct((n, 1), jnp.float32),
    )(x, W, a.reshape(-1, 1))


def _edge_l1_kernel(ad_ref, as_ref, bs_ref, ea_ref, wge_ref, wme_ref, msg_ref):
    ea = ea_ref[...]
    gate = _sig(ad_ref[...] + as_ref[...] + _dot(ea, wge_ref[...]))
    msg_ref[...] = gate * _lrelu(bs_ref[...] + _dot(ea, wme_ref[...]))


def _edge_l1(ad_d, as_s, bs_s, ea_pad, WgE, WmE):
    e, h = ad_d.shape
    ke = ea_pad.shape[1]
    be = _blk(e, 4000)
    return pl.pallas_call(
        _edge_l1_kernel,
        grid=(e // be,),
        in_specs=[
            pl.BlockSpec((be, h), lambda i: (i, 0)),
            pl.BlockSpec((be, h), lambda i: (i, 0)),
            pl.BlockSpec((be, h), lambda i: (i, 0)),
            pl.BlockSpec((be, ke), lambda i: (i, 0)),
            pl.BlockSpec((ke, h), lambda i: (0, 0)),
            pl.BlockSpec((ke, h), lambda i: (0, 0)),
        ],
        out_specs=pl.BlockSpec((be, h), lambda i: (i, 0)),
        out_shape=jax.ShapeDtypeStruct((e, h), jnp.float32),
    )(ad_d, as_s, bs_s, ea_pad, WgE, WmE)


def _edge_l2_kernel(hs_ref, ls_ref, ld_ref, wmsg_ref, e_ref):
    ew = jnp.exp(_lrelu(ls_ref[...] + ld_ref[...]))
    e_ref[...] = ew
    wmsg_ref[...] = ew * hs_ref[...]


def _edge_l2(hs_s, ls_s, ld_d):
    e, h = hs_s.shape
    be = _blk(e, 4000)
    return pl.pallas_call(
        _edge_l2_kernel,
        grid=(e // be,),
        in_specs=[
            pl.BlockSpec((be, h), lambda i: (i, 0)),
            pl.BlockSpec((be, 1), lambda i: (i, 0)),
            pl.BlockSpec((be, 1), lambda i: (i, 0)),
        ],
        out_specs=[
            pl.BlockSpec((be, h), lambda i: (i, 0)),
            pl.BlockSpec((be, 1), lambda i: (i, 0)),
        ],
        out_shape=[
            jax.ShapeDtypeStruct((e, h), jnp.float32),
            jax.ShapeDtypeStruct((e, 1), jnp.float32),
        ],
    )(hs_s, ls_s, ld_d)


def _gru_kernel(a_ref, h_ref, wir_ref, wiz_ref, win_ref, whr_ref, whz_ref,
                whn_ref, bir_ref, biz_ref, bin_ref, bhr_ref, bhz_ref,
                bhn_ref, o_ref):
    a = a_ref[...]
    h = h_ref[...]
    r = _sig(_dot(a, wir_ref[...]) + bir_ref[...] + _dot(h, whr_ref[...]) + bhr_ref[...])
    z = _sig(_dot(a, wiz_ref[...]) + biz_ref[...] + _dot(h, whz_ref[...]) + bhz_ref[...])
    nn = jnp.tanh(_dot(a, win_ref[...]) + bin_ref[...] + r * (_dot(h, whn_ref[...]) + bhn_ref[...]))
    o_ref[...] = (1.0 - z) * nn + z * h


def _gru_div_kernel(num_ref, den_ref, h_ref, wir_ref, wiz_ref, win_ref,
                    whr_ref, whz_ref, whn_ref, bir_ref, biz_ref, bin_ref,
                    bhr_ref, bhz_ref, bhn_ref, o_ref):
    a = num_ref[...] / (den_ref[...] + 1e-16)
    h = h_ref[...]
    r = _sig(_dot(a, wir_ref[...]) + bir_ref[...] + _dot(h, whr_ref[...]) + bhr_ref[...])
    z = _sig(_dot(a, wiz_ref[...]) + biz_ref[...] + _dot(h, whz_ref[...]) + bhz_ref[...])
    nn = jnp.tanh(_dot(a, win_ref[...]) + bin_ref[...] + r * (_dot(h, whn_ref[...]) + bhn_ref[...]))
    o_ref[...] = (1.0 - z) * nn + z * h


def _split_gru_w(Wgi, Wgh, bgi, bgh):
    h = Wgi.shape[0]
    ws = [Wgi[:, :h], Wgi[:, h:2 * h], Wgi[:, 2 * h:],
          Wgh[:, :h], Wgh[:, h:2 * h], Wgh[:, 2 * h:]]
    bs = [bgi[:h], bgi[h:2 * h], bgi[2 * h:],
          bgh[:h], bgh[h:2 * h], bgh[2 * h:]]
    return ws, [b.reshape(1, -1) for b in bs]


def _gru(a, h, Wgi, Wgh, bgi, bgh, den=None):
    n, hh = a.shape
    bn = _blk(n, 5000)
    ws, bs = _split_gru_w(Wgi, Wgh, bgi, bgh)
    wspec = [pl.BlockSpec((hh, hh), lambda i: (0, 0))] * 6
    bspec = [pl.BlockSpec((1, hh), lambda i: (0, 0))] * 6
    row = pl.BlockSpec((bn, hh), lambda i: (i, 0))
    col = pl.BlockSpec((bn, 1), lambda i: (i, 0))
    if den is None:
        return pl.pallas_call(
            _gru_kernel,
            grid=(n // bn,),
            in_specs=[row, row] + wspec + bspec,
            out_specs=row,
            out_shape=jax.ShapeDtypeStruct((n, hh), jnp.float32),
        )(a, h, *ws, *bs)
    return pl.pallas_call(
        _gru_div_kernel,
        grid=(n // bn,),
        in_specs=[row, col, row] + wspec + bspec,
        out_specs=row,
        out_shape=jax.ShapeDtypeStruct((n, hh), jnp.float32),
    )(a, den, h, *ws, *bs)


def _add_relu_kernel(a_ref, b_ref, o_ref):
    o_ref[...] = jnp.maximum(a_ref[...] + b_ref[...], 0.0)


def _add_relu(a, b):
    n, h = a.shape
    bn = _blk(n, 5000)
    row = pl.BlockSpec((bn, h), lambda i: (i, 0))
    return pl.pallas_call(
        _add_relu_kernel,
        grid=(n // bn,),
        in_specs=[row, row],
        out_specs=row,
        out_shape=jax.ShapeDtypeStruct((n, h), jnp.float32),
    )(a, b)


def _pool0_kernel(g_ref, brow_ref, o_ref):
    i = pl.program_id(0)

    @pl.when(i == 0)
    def _():
        o_ref[...] = jnp.zeros_like(o_ref)

    bn = g_ref.shape[0]
    brow = brow_ref[...].reshape(1, bn)
    oht = jnp.where(
        jax.lax.broadcasted_iota(jnp.int32, (G_SEG, bn), 0) == brow,
        1.0, 0.0)
    o_ref[...] += _dot(oht, g_ref[...])

    @pl.when(i == pl.num_programs(0) - 1)
    def _():
        o_ref[...] = jnp.maximum(o_ref[...], 0.0)


def _pool0(g, batch):
    n, h = g.shape
    bn = _blk(n, 5000)
    batch_row = batch.reshape(n // bn, 1, bn)
    return pl.pallas_call(
        _pool0_kernel,
        grid=(n // bn,),
        in_specs=[
            pl.BlockSpec((bn, h), lambda i: (i, 0)),
            pl.BlockSpec((1, 1, bn), lambda i: (i, 0, 0)),
        ],
        out_specs=pl.BlockSpec((G_SEG, h), lambda i: (0, 0)),
        out_shape=jax.ShapeDtypeStruct((G_SEG, h), jnp.float32),
    )(g, batch_row)


def _molpool_kernel(bcol_ref, brow_ref, ls_ref, hs_ref, xmol_ref, ad_ref,
                    num_ref, den_ref):
    i = pl.program_id(0)

    @pl.when(i == 0)
    def _():
        num_ref[...] = jnp.zeros_like(num_ref)
        den_ref[...] = jnp.zeros_like(den_ref)

    bn = hs_ref.shape[0]
    t = _dot(xmol_ref[...], ad_ref[...])
    ohg = jnp.where(
        jax.lax.broadcasted_iota(jnp.int32, (bn, G_SEG), 1) == bcol_ref[...],
        1.0, 0.0)
    td = _dot(ohg, t)
    e = jnp.exp(_lrelu(ls_ref[...] + td))
    brow = brow_ref[...].reshape(1, bn)
    oht = jnp.where(
        jax.lax.broadcasted_iota(jnp.int32, (G_SEG, bn), 0) == brow,
        1.0, 0.0)
    num_ref[...] += _dot(oht, e * hs_ref[...])
    den_ref[...] += _dot(oht, e)


def _molpool(batch, ls, hs, x_mol, ad):
    n, h = hs.shape
    bn = _blk(n, 5000)
    batch_col = batch.reshape(n, 1)
    batch_row = batch.reshape(n // bn, 1, bn)
    return pl.pallas_call(
        _molpool_kernel,
        grid=(n // bn,),
        in_specs=[
            pl.BlockSpec((bn, 1), lambda i: (i, 0)),
            pl.BlockSpec((1, 1, bn), lambda i: (i, 0, 0)),
            pl.BlockSpec((bn, 1), lambda i: (i, 0)),
            pl.BlockSpec((bn, h), lambda i: (i, 0)),
            pl.BlockSpec((G_SEG, h), lambda i: (0, 0)),
            pl.BlockSpec((h, 1), lambda i: (0, 0)),
        ],
        out_specs=[
            pl.BlockSpec((G_SEG, h), lambda i: (0, 0)),
            pl.BlockSpec((G_SEG, 1), lambda i: (0, 0)),
        ],
        out_shape=[
            jax.ShapeDtypeStruct((G_SEG, h), jnp.float32),
            jax.ShapeDtypeStruct((G_SEG, 1), jnp.float32),
        ],
    )(batch_col, batch_row, ls, hs, x_mol, ad.reshape(-1, 1))


def _head_kernel(mpa_ref, mla_ref, wpa_ref, bpa_ref, wla_ref, bla_ref,
                 w1a_ref, w1b_ref, b1_ref, w2_ref, b2_ref, y_ref):
    y_pa = _dot(mpa_ref[...], wpa_ref[...]) + bpa_ref[...]
    y_la = _dot(mla_ref[...], wla_ref[...]) + bla_ref[...]
    z = jnp.maximum(_dot(y_pa, w1a_ref[...]) + _dot(y_la, w1b_ref[...]) + b1_ref[...], 0.0)
    y_ref[...] = _dot(z, w2_ref[...]) + b2_ref[...]


def _head(m_pa, m_la, lin_pa_W, lin_pa_b, lin_la_W, lin_la_b, mlp_W1, mlp_b1,
          mlp_W2, mlp_b2):
    h = m_pa.shape[1]
    return pl.pallas_call(
        _head_kernel,
        out_shape=jax.ShapeDtypeStruct((G_SEG, 1), jnp.float32),
    )(m_pa, m_la, lin_pa_W, lin_pa_b.reshape(1, -1), lin_la_W,
      lin_la_b.reshape(1, -1), mlp_W1[:h], mlp_W1[h:], mlp_b1.reshape(1, -1),
      mlp_W2, mlp_b2.reshape(1, -1))


def _gate_conv(xs, xd, ei, ea_pad, Wg, bg, Wm, bm, Wi0, bi0, Wgi, Wgh, bgi, bgh):
    f = xs.shape[1]
    s, d = ei[0], ei[1]
    as_t, bs_t = _dual_mm(xs, Wg[f:2 * f], bg, Wm[:f], bm, act2=False)
    ad_t, h0 = _dual_mm(xd, Wg[:f], jnp.zeros_like(bg), Wi0, bi0, act2=True)
    msg = _edge_l1(ad_t[d], as_t[s], bs_t[s], ea_pad, Wg[2 * f:], Wm[f:])
    agg = jax.ops.segment_sum(msg, d, num_segments=xd.shape[0])
    return _gru(agg, h0, Wgi, Wgh, bgi, bgh)


def _gat_conv(xs, xd, ei, Ws, Wd, a_s, a_d, Wgi, Wgh, bgi, bgh):
    s, d = ei[0], ei[1]
    hs, ls = _proj_attn(xs, Ws, a_s)
    ld = _attn_scalar(xd, Wd, a_d)
    wmsg, ew = _edge_l2(hs[s], ls[s], ld[d])
    num = jax.ops.segment_sum(wmsg, d, num_segments=xd.shape[0])
    den = jax.ops.segment_sum(ew, d, num_segments=xd.shape[0])
    return _gru(num, xd, Wgi, Wgh, bgi, bgh, den=den)


def _pad_ea(ea):
    e, f = ea.shape
    return jnp.concatenate([ea, jnp.zeros((e, 8 - f), jnp.float32)], axis=1)


def _pad_we(W):
    f, h = W.shape
    return jnp.concatenate([W, jnp.zeros((8 - f, h), jnp.float32)], axis=0)


def kernel(x_pa, x_la, edge_index_pp, edge_attr_pp, edge_index_ll, edge_attr_ll, edge_index_lp, edge_attr_lp, edge_index_pl, edge_attr_pl, batch_pa, batch_la, l1_pp_Wg, l1_pp_bg, l1_pp_Wm, l1_pp_bm, l1_pp_Wi0, l1_pp_bi0, l1_pp_Wgi, l1_pp_Wgh, l1_pp_bgi, l1_pp_bgh, l1_ll_Wg, l1_ll_bg, l1_ll_Wm, l1_ll_bm, l1_ll_Wi0, l1_ll_bi0, l1_ll_Wgi, l1_ll_Wgh, l1_ll_bgi, l1_ll_bgh, l1_lp_Wg, l1_lp_bg, l1_lp_Wm, l1_lp_bm, l1_lp_Wi0, l1_lp_bi0, l1_lp_Wgi, l1_lp_Wgh, l1_lp_bgi, l1_lp_bgh, l1_pl_Wg, l1_pl_bg, l1_pl_Wm, l1_pl_bm, l1_pl_Wi0, l1_pl_bi0, l1_pl_Wgi, l1_pl_Wgh, l1_pl_bgi, l1_pl_bgh, l2_pp_Ws, l2_pp_Wd, l2_pp_as, l2_pp_ad, l2_pp_Wgi, l2_pp_Wgh, l2_pp_bgi, l2_pp_bgh, l2_ll_Ws, l2_ll_Wd, l2_ll_as, l2_ll_ad, l2_ll_Wgi, l2_ll_Wgh, l2_ll_bgi, l2_ll_bgh, l2_lp_Ws, l2_lp_Wd, l2_lp_as, l2_lp_ad, l2_lp_Wgi, l2_lp_Wgh, l2_lp_bgi, l2_lp_bgh, l2_pl_Ws, l2_pl_Wd, l2_pl_as, l2_pl_ad, l2_pl_Wgi, l2_pl_Wgh, l2_pl_bgi, l2_pl_bgh, mol_pa_Ws, mol_pa_as, mol_pa_ad, mol_pa_Wgi, mol_pa_Wgh, mol_pa_bgi, mol_pa_bgh, mol_la_Ws, mol_la_as, mol_la_ad, mol_la_Wgi, mol_la_Wgh, mol_la_bgi, mol_la_bgh, lin_pa_W, lin_pa_b, lin_la_W, lin_la_b, mlp_W1, mlp_b1, mlp_W2, mlp_b2):
    ea = {"pp": _pad_ea(edge_attr_pp), "ll": _pad_ea(edge_attr_ll),
          "lp": _pad_ea(edge_attr_lp), "pl": _pad_ea(edge_attr_pl)}
    ei = {"pp": edge_index_pp, "ll": edge_index_ll,
          "lp": edge_index_lp, "pl": edge_index_pl}
    f = x_pa.shape[1]

    def l1w(r, Wg, bg, Wm, bm, Wi0, bi0, Wgi, Wgh, bgi, bgh):
        Wg_e = _pad_we(Wg[2 * f:])
        Wm_e = _pad_we(Wm[f:])
        Wg2 = jnp.concatenate([Wg[:2 * f], Wg_e], axis=0)
        Wm2 = jnp.concatenate([Wm[:f], Wm_e], axis=0)
        return (Wg2, bg, Wm2, bm, Wi0, bi0, Wgi, Wgh, bgi, bgh)

    p1 = {
        "pp": l1w("pp", l1_pp_Wg, l1_pp_bg, l1_pp_Wm, l1_pp_bm, l1_pp_Wi0, l1_pp_bi0, l1_pp_Wgi, l1_pp_Wgh, l1_pp_bgi, l1_pp_bgh),
        "ll": l1w("ll", l1_ll_Wg, l1_ll_bg, l1_ll_Wm, l1_ll_bm, l1_ll_Wi0, l1_ll_bi0, l1_ll_Wgi, l1_ll_Wgh, l1_ll_bgi, l1_ll_bgh),
        "lp": l1w("lp", l1_lp_Wg, l1_lp_bg, l1_lp_Wm, l1_lp_bm, l1_lp_Wi0, l1_lp_bi0, l1_lp_Wgi, l1_lp_Wgh, l1_lp_bgi, l1_lp_bgh),
        "pl": l1w("pl", l1_pl_Wg, l1_pl_bg, l1_pl_Wm, l1_pl_bm, l1_pl_Wi0, l1_pl_bi0, l1_pl_Wgi, l1_pl_Wgh, l1_pl_bgi, l1_pl_bgh),
    }
    h_pa = _add_relu(_gate_conv(x_pa, x_pa, ei["pp"], ea["pp"], *p1["pp"]),
                     _gate_conv(x_la, x_pa, ei["lp"], ea["lp"], *p1["lp"]))
    h_la = _add_relu(_gate_conv(x_la, x_la, ei["ll"], ea["ll"], *p1["ll"]),
                     _gate_conv(x_pa, x_la, ei["pl"], ea["pl"], *p1["pl"]))

    g_pa = _add_relu(
        _gat_conv(h_pa, h_pa, ei["pp"], l2_pp_Ws, l2_pp_Wd, l2_pp_as, l2_pp_ad, l2_pp_Wgi, l2_pp_Wgh, l2_pp_bgi, l2_pp_bgh),
        _gat_conv(h_la, h_pa, ei["lp"], l2_lp_Ws, l2_lp_Wd, l2_lp_as, l2_lp_ad, l2_lp_Wgi, l2_lp_Wgh, l2_lp_bgi, l2_lp_bgh))
    g_la = _add_relu(
        _gat_conv(h_la, h_la, ei["ll"], l2_ll_Ws, l2_ll_Wd, l2_ll_as, l2_ll_ad, l2_ll_Wgi, l2_ll_Wgh, l2_ll_bgi, l2_ll_bgh),
        _gat_conv(h_pa, h_la, ei["pl"], l2_pl_Ws, l2_pl_Wd, l2_pl_as, l2_pl_ad, l2_pl_Wgi, l2_pl_Wgh, l2_pl_bgi, l2_pl_bgh))

    bpa = batch_pa.astype(jnp.int32)
    bla = batch_la.astype(jnp.int32)

    m_pa = _pool0(g_pa, bpa)
    m_la = _pool0(g_la, bla)

    hs_pa, ls_pa = _proj_attn(g_pa, mol_pa_Ws, mol_pa_as)
    hs_la, ls_la = _proj_attn(g_la, mol_la_Ws, mol_la_as)
    for _ in range(2):
        num, den = _molpool(bpa, ls_pa, hs_pa, m_pa, mol_pa_ad)
        m_pa = _gru(num, m_pa, mol_pa_Wgi, mol_pa_Wgh, mol_pa_bgi, mol_pa_bgh, den=den)
        num, den = _molpool(bla, ls_la, hs_la, m_la, mol_la_ad)
        m_la = _gru(num, m_la, mol_la_Wgi, mol_la_Wgh, mol_la_bgi, mol_la_bgh, den=den)

    return _head(m_pa, m_la, lin_pa_W, lin_pa_b, lin_la_W, lin_la_b,
                 mlp_W1, mlp_b1, mlp_W2, mlp_b2)


# speedup vs baseline: 1.0875x; 1.0875x over previous
import functools

import jax
import jax.numpy as jnp
from jax.experimental import pallas as pl

G_SEG = 256
_SLOPE = 0.2


def _sig(x):
    return 1.0 / (1.0 + jnp.exp(-x))


def _lrelu(x):
    return jnp.where(x > 0, x, _SLOPE * x)


def _blk(n, target):
    b = min(n, target)
    while n % b or b % 8:
        b -= 1
    return b


def _dot(a, b):
    return jnp.dot(a, b, preferred_element_type=jnp.float32,
                   precision=jax.lax.Precision.HIGHEST)


def _dual_mm_kernel(x_ref, w1_ref, b1_ref, w2_ref, b2_ref, o1_ref, o2_ref, *, act2):
    x = x_ref[...]
    o1_ref[...] = _dot(x, w1_ref[...]) + b1_ref[...]
    y = _dot(x, w2_ref[...]) + b2_ref[...]
    o2_ref[...] = _lrelu(y) if act2 else y


def _dual_mm(x, W1, b1, W2, b2, act2):
    n, k = x.shape
    bn = _blk(n, 5000)
    h1 = W1.shape[1]
    h2 = W2.shape[1]
    return pl.pallas_call(
        functools.partial(_dual_mm_kernel, act2=act2),
        grid=(n // bn,),
        in_specs=[
            pl.BlockSpec((bn, k), lambda i: (i, 0)),
            pl.BlockSpec((k, h1), lambda i: (0, 0)),
            pl.BlockSpec((1, h1), lambda i: (0, 0)),
            pl.BlockSpec((k, h2), lambda i: (0, 0)),
            pl.BlockSpec((1, h2), lambda i: (0, 0)),
        ],
        out_specs=[
            pl.BlockSpec((bn, h1), lambda i: (i, 0)),
            pl.BlockSpec((bn, h2), lambda i: (i, 0)),
        ],
        out_shape=[
            jax.ShapeDtypeStruct((n, h1), jnp.float32),
            jax.ShapeDtypeStruct((n, h2), jnp.float32),
        ],
    )(x, W1, b1.reshape(1, -1), W2, b2.reshape(1, -1))


def _proj_attn_kernel(x_ref, w_ref, a_ref, hs_ref, ls_ref):
    hs = _dot(x_ref[...], w_ref[...])
    hs_ref[...] = hs
    ls_ref[...] = _dot(hs, a_ref[...])


def _proj_attn(x, W, a):
    n, k = x.shape
    h = W.shape[1]
    bn = _blk(n, 5000)
    return pl.pallas_call(
        _proj_attn_kernel,
        grid=(n // bn,),
        in_specs=[
            pl.BlockSpec((bn, k), lambda i: (i, 0)),
            pl.BlockSpec((k, h), lambda i: (0, 0)),
            pl.BlockSpec((h, 1), lambda i: (0, 0)),
        ],
        out_specs=[
            pl.BlockSpec((bn, h), lambda i: (i, 0)),
            pl.BlockSpec((bn, 1), lambda i: (i, 0)),
        ],
        out_shape=[
            jax.ShapeDtypeStruct((n, h), jnp.float32),
            jax.ShapeDtypeStruct((n, 1), jnp.float32),
        ],
    )(x, W, a.reshape(-1, 1))


def _attn_scalar_kernel(x_ref, w_ref, a_ref, t_ref):
    t_ref[...] = _dot(_dot(x_ref[...], w_ref[...]), a_ref[...])


def _attn_scalar(x, W, a):
    n, k = x.shape
    h = W.shape[1]
    bn = _blk(n, 5000)
    return pl.pallas_call(
        _attn_scalar_kernel,
        grid=(n // bn,),
        in_specs=[
            pl.BlockSpec((bn, k), lambda i: (i, 0)),
            pl.BlockSpec((k, h), lambda i: (0, 0)),
            pl.BlockSpec((h, 1), lambda i: (0, 0)),
        ],
        out_specs=pl.BlockSpec((bn, 1), lambda i: (i, 0)),
        out_shape=jax.ShapeDtypeStruct((n, 1), jnp.float32),
    )(x, W, a.reshape(-1, 1))


def _edge_l1_kernel(xd_ref, xs_ref, ea_ref, wgd_ref, wgs_ref, wms_ref,
                    wge_ref, wme_ref, bg_ref, bm_ref, msg_ref):
    xd = xd_ref[...]
    xs = xs_ref[...]
    ea = ea_ref[...]
    gate = _sig(_dot(xd, wgd_ref[...]) + _dot(xs, wgs_ref[...]) +
                _dot(ea, wge_ref[...]) + bg_ref[...])
    msg_ref[...] = gate * _lrelu(_dot(xs, wms_ref[...]) +
                                 _dot(ea, wme_ref[...]) + bm_ref[...])


def _edge_l1(xd_g, xs_g, ea_pad, Wgd, Wgs, Wms, WgE, WmE, bg, bm):
    e, kf = xd_g.shape
    ke = ea_pad.shape[1]
    h = Wgd.shape[1]
    be = _blk(e, 4000)
    row40 = pl.BlockSpec((be, kf), lambda i: (i, 0))
    wsp = lambda k: pl.BlockSpec((k, h), lambda i: (0, 0))
    return pl.pallas_call(
        _edge_l1_kernel,
        grid=(e // be,),
        in_specs=[
            row40, row40,
            pl.BlockSpec((be, ke), lambda i: (i, 0)),
            wsp(kf), wsp(kf), wsp(kf), wsp(ke), wsp(ke),
            pl.BlockSpec((1, h), lambda i: (0, 0)),
            pl.BlockSpec((1, h), lambda i: (0, 0)),
        ],
        out_specs=pl.BlockSpec((be, h), lambda i: (i, 0)),
        out_shape=jax.ShapeDtypeStruct((e, h), jnp.float32),
    )(xd_g, xs_g, ea_pad, Wgd, Wgs, Wms, WgE, WmE,
      bg.reshape(1, -1), bm.reshape(1, -1))


def _edge_l2_kernel(hs_ref, ls_ref, ld_ref, out_ref):
    ew = jnp.exp(_lrelu(ls_ref[...] + ld_ref[...]))
    be = hs_ref.shape[0]
    out_ref[...] = jnp.concatenate(
        [ew * hs_ref[...], ew, jnp.zeros((be, 7), jnp.float32)], axis=1)


def _edge_l2(hs_s, ls_s, ld_d):
    e, h = hs_s.shape
    be = _blk(e, 4000)
    return pl.pallas_call(
        _edge_l2_kernel,
        grid=(e // be,),
        in_specs=[
            pl.BlockSpec((be, h), lambda i: (i, 0)),
            pl.BlockSpec((be, 1), lambda i: (i, 0)),
            pl.BlockSpec((be, 1), lambda i: (i, 0)),
        ],
        out_specs=pl.BlockSpec((be, h + 8), lambda i: (i, 0)),
        out_shape=jax.ShapeDtypeStruct((e, h + 8), jnp.float32),
    )(hs_s, ls_s, ld_d)


def _gru_kernel(a_ref, h_ref, wir_ref, wiz_ref, win_ref, whr_ref, whz_ref,
                whn_ref, bir_ref, biz_ref, bin_ref, bhr_ref, bhz_ref,
                bhn_ref, o_ref):
    a = a_ref[...]
    h = h_ref[...]
    r = _sig(_dot(a, wir_ref[...]) + bir_ref[...] + _dot(h, whr_ref[...]) + bhr_ref[...])
    z = _sig(_dot(a, wiz_ref[...]) + biz_ref[...] + _dot(h, whz_ref[...]) + bhz_ref[...])
    nn = jnp.tanh(_dot(a, win_ref[...]) + bin_ref[...] + r * (_dot(h, whn_ref[...]) + bhn_ref[...]))
    o_ref[...] = (1.0 - z) * nn + z * h


def _gru_div_kernel(num_ref, den_ref, h_ref, wir_ref, wiz_ref, win_ref,
                    whr_ref, whz_ref, whn_ref, bir_ref, biz_ref, bin_ref,
                    bhr_ref, bhz_ref, bhn_ref, o_ref):
    a = num_ref[...] / (den_ref[...] + 1e-16)
    h = h_ref[...]
    r = _sig(_dot(a, wir_ref[...]) + bir_ref[...] + _dot(h, whr_ref[...]) + bhr_ref[...])
    z = _sig(_dot(a, wiz_ref[...]) + biz_ref[...] + _dot(h, whz_ref[...]) + bhz_ref[...])
    nn = jnp.tanh(_dot(a, win_ref[...]) + bin_ref[...] + r * (_dot(h, whn_ref[...]) + bhn_ref[...]))
    o_ref[...] = (1.0 - z) * nn + z * h


def _split_gru_w(Wgi, Wgh, bgi, bgh):
    h = Wgi.shape[0]
    ws = [Wgi[:, :h], Wgi[:, h:2 * h], Wgi[:, 2 * h:],
          Wgh[:, :h], Wgh[:, h:2 * h], Wgh[:, 2 * h:]]
    bs = [bgi[:h], bgi[h:2 * h], bgi[2 * h:],
          bgh[:h], bgh[h:2 * h], bgh[2 * h:]]
    return ws, [b.reshape(1, -1) for b in bs]


def _gru(a, h, Wgi, Wgh, bgi, bgh, den=None):
    n, hh = a.shape
    bn = _blk(n, 5000)
    ws, bs = _split_gru_w(Wgi, Wgh, bgi, bgh)
    wspec = [pl.BlockSpec((hh, hh), lambda i: (0, 0))] * 6
    bspec = [pl.BlockSpec((1, hh), lambda i: (0, 0))] * 6
    row = pl.BlockSpec((bn, hh), lambda i: (i, 0))
    col = pl.BlockSpec((bn, 1), lambda i: (i, 0))
    if den is None:
        return pl.pallas_call(
            _gru_kernel,
            grid=(n // bn,),
            in_specs=[row, row] + wspec + bspec,
            out_specs=row,
            out_shape=jax.ShapeDtypeStruct((n, hh), jnp.float32),
        )(a, h, *ws, *bs)
    return pl.pallas_call(
        _gru_div_kernel,
        grid=(n // bn,),
        in_specs=[row, col, row] + wspec + bspec,
        out_specs=row,
        out_shape=jax.ShapeDtypeStruct((n, hh), jnp.float32),
    )(a, den, h, *ws, *bs)


def _add_relu_kernel(a_ref, b_ref, o_ref):
    o_ref[...] = jnp.maximum(a_ref[...] + b_ref[...], 0.0)


def _add_relu(a, b):
    n, h = a.shape
    bn = _blk(n, 5000)
    row = pl.BlockSpec((bn, h), lambda i: (i, 0))
    return pl.pallas_call(
        _add_relu_kernel,
        grid=(n // bn,),
        in_specs=[row, row],
        out_specs=row,
        out_shape=jax.ShapeDtypeStruct((n, h), jnp.float32),
    )(a, b)


def _pool0_kernel(g_ref, brow_ref, o_ref):
    i = pl.program_id(0)

    @pl.when(i == 0)
    def _():
        o_ref[...] = jnp.zeros_like(o_ref)

    bn = g_ref.shape[0]
    brow = brow_ref[...].reshape(1, bn)
    oht = jnp.where(
        jax.lax.broadcasted_iota(jnp.int32, (G_SEG, bn), 0) == brow,
        1.0, 0.0)
    o_ref[...] += _dot(oht, g_ref[...])

    @pl.when(i == pl.num_programs(0) - 1)
    def _():
        o_ref[...] = jnp.maximum(o_ref[...], 0.0)


def _pool0(g, batch):
    n, h = g.shape
    bn = _blk(n, 5000)
    batch_row = batch.reshape(n // bn, 1, bn)
    return pl.pallas_call(
        _pool0_kernel,
        grid=(n // bn,),
        in_specs=[
            pl.BlockSpec((bn, h), lambda i: (i, 0)),
            pl.BlockSpec((1, 1, bn), lambda i: (i, 0, 0)),
        ],
        out_specs=pl.BlockSpec((G_SEG, h), lambda i: (0, 0)),
        out_shape=jax.ShapeDtypeStruct((G_SEG, h), jnp.float32),
    )(g, batch_row)


def _molpool_kernel(bcol_ref, brow_ref, ls_ref, hs_ref, xmol_ref, ad_ref,
                    num_ref, den_ref):
    i = pl.program_id(0)

    @pl.when(i == 0)
    def _():
        num_ref[...] = jnp.zeros_like(num_ref)
        den_ref[...] = jnp.zeros_like(den_ref)

    bn = hs_ref.shape[0]
    t = _dot(xmol_ref[...], ad_ref[...])
    ohg = jnp.where(
        jax.lax.broadcasted_iota(jnp.int32, (bn, G_SEG), 1) == bcol_ref[...],
        1.0, 0.0)
    td = _dot(ohg, t)
    e = jnp.exp(_lrelu(ls_ref[...] + td))
    brow = brow_ref[...].reshape(1, bn)
    oht = jnp.where(
        jax.lax.broadcasted_iota(jnp.int32, (G_SEG, bn), 0) == brow,
        1.0, 0.0)
    num_ref[...] += _dot(oht, e * hs_ref[...])
    den_ref[...] += _dot(oht, e)


def _molpool(batch, ls, hs, x_mol, ad):
    n, h = hs.shape
    bn = _blk(n, 5000)
    batch_col = batch.reshape(n, 1)
    batch_row = batch.reshape(n // bn, 1, bn)
    return pl.pallas_call(
        _molpool_kernel,
        grid=(n // bn,),
        in_specs=[
            pl.BlockSpec((bn, 1), lambda i: (i, 0)),
            pl.BlockSpec((1, 1, bn), lambda i: (i, 0, 0)),
            pl.BlockSpec((bn, 1), lambda i: (i, 0)),
            pl.BlockSpec((bn, h), lambda i: (i, 0)),
            pl.BlockSpec((G_SEG, h), lambda i: (0, 0)),
            pl.BlockSpec((h, 1), lambda i: (0, 0)),
        ],
        out_specs=[
            pl.BlockSpec((G_SEG, h), lambda i: (0, 0)),
            pl.BlockSpec((G_SEG, 1), lambda i: (0, 0)),
        ],
        out_shape=[
            jax.ShapeDtypeStruct((G_SEG, h), jnp.float32),
            jax.ShapeDtypeStruct((G_SEG, 1), jnp.float32),
        ],
    )(batch_col, batch_row, ls, hs, x_mol, ad.reshape(-1, 1))


def _head_kernel(mpa_ref, mla_ref, wpa_ref, bpa_ref, wla_ref, bla_ref,
                 w1a_ref, w1b_ref, b1_ref, w2_ref, b2_ref, y_ref):
    y_pa = _dot(mpa_ref[...], wpa_ref[...]) + bpa_ref[...]
    y_la = _dot(mla_ref[...], wla_ref[...]) + bla_ref[...]
    z = jnp.maximum(_dot(y_pa, w1a_ref[...]) + _dot(y_la, w1b_ref[...]) + b1_ref[...], 0.0)
    y_ref[...] = _dot(z, w2_ref[...]) + b2_ref[...]


def _head(m_pa, m_la, lin_pa_W, lin_pa_b, lin_la_W, lin_la_b, mlp_W1, mlp_b1,
          mlp_W2, mlp_b2):
    h = m_pa.shape[1]
    return pl.pallas_call(
        _head_kernel,
        out_shape=jax.ShapeDtypeStruct((G_SEG, 1), jnp.float32),
    )(m_pa, m_la, lin_pa_W, lin_pa_b.reshape(1, -1), lin_la_W,
      lin_la_b.reshape(1, -1), mlp_W1[:h], mlp_W1[h:], mlp_b1.reshape(1, -1),
      mlp_W2, mlp_b2.reshape(1, -1))


def _mm_act_kernel(x_ref, w_ref, b_ref, o_ref):
    o_ref[...] = _lrelu(_dot(x_ref[...], w_ref[...]) + b_ref[...])


def _mm_act(x, W, b):
    n, k = x.shape
    h = W.shape[1]
    bn = _blk(n, 5000)
    return pl.pallas_call(
        _mm_act_kernel,
        grid=(n // bn,),
        in_specs=[
            pl.BlockSpec((bn, k), lambda i: (i, 0)),
            pl.BlockSpec((k, h), lambda i: (0, 0)),
            pl.BlockSpec((1, h), lambda i: (0, 0)),
        ],
        out_specs=pl.BlockSpec((bn, h), lambda i: (i, 0)),
        out_shape=jax.ShapeDtypeStruct((n, h), jnp.float32),
    )(x, W, b.reshape(1, -1))


def _gate_conv(xs_pad, xd_pad, xd, ei, ea_pad, Wgd, Wgs, Wms, WgE, WmE,
               bg, bm, Wi0, bi0, Wgi, Wgh, bgi, bgh):
    s, d = ei[0], ei[1]
    h0 = _mm_act(xd, Wi0, bi0)
    msg = _edge_l1(xd_pad[d], xs_pad[s], ea_pad, Wgd, Wgs, Wms, WgE, WmE, bg, bm)
    agg = jax.ops.segment_sum(msg, d, num_segments=xd.shape[0])
    return _gru(agg, h0, Wgi, Wgh, bgi, bgh)


def _gat_conv(xs, xd, ei, Ws, Wd, a_s, a_d, Wgi, Wgh, bgi, bgh):
    s, d = ei[0], ei[1]
    hs, ls = _proj_attn(xs, Ws, a_s)
    ld = _attn_scalar(xd, Wd, a_d)
    comb = _edge_l2(hs[s], ls[s], ld[d])
    seg = jax.ops.segment_sum(comb, d, num_segments=xd.shape[0])
    return _gru(seg[:, :64], xd, Wgi, Wgh, bgi, bgh, den=seg[:, 64:65])


def _pad_cols(x, to):
    n, f = x.shape
    return jnp.concatenate([x, jnp.zeros((n, to - f), jnp.float32)], axis=1)


def _pad_rows(W, to):
    f, h = W.shape
    return jnp.concatenate([W, jnp.zeros((to - f, h), jnp.float32)], axis=0)


def kernel(x_pa, x_la, edge_index_pp, edge_attr_pp, edge_index_ll, edge_attr_ll, edge_index_lp, edge_attr_lp, edge_index_pl, edge_attr_pl, batch_pa, batch_la, l1_pp_Wg, l1_pp_bg, l1_pp_Wm, l1_pp_bm, l1_pp_Wi0, l1_pp_bi0, l1_pp_Wgi, l1_pp_Wgh, l1_pp_bgi, l1_pp_bgh, l1_ll_Wg, l1_ll_bg, l1_ll_Wm, l1_ll_bm, l1_ll_Wi0, l1_ll_bi0, l1_ll_Wgi, l1_ll_Wgh, l1_ll_bgi, l1_ll_bgh, l1_lp_Wg, l1_lp_bg, l1_lp_Wm, l1_lp_bm, l1_lp_Wi0, l1_lp_bi0, l1_lp_Wgi, l1_lp_Wgh, l1_lp_bgi, l1_lp_bgh, l1_pl_Wg, l1_pl_bg, l1_pl_Wm, l1_pl_bm, l1_pl_Wi0, l1_pl_bi0, l1_pl_Wgi, l1_pl_Wgh, l1_pl_bgi, l1_pl_bgh, l2_pp_Ws, l2_pp_Wd, l2_pp_as, l2_pp_ad, l2_pp_Wgi, l2_pp_Wgh, l2_pp_bgi, l2_pp_bgh, l2_ll_Ws, l2_ll_Wd, l2_ll_as, l2_ll_ad, l2_ll_Wgi, l2_ll_Wgh, l2_ll_bgi, l2_ll_bgh, l2_lp_Ws, l2_lp_Wd, l2_lp_as, l2_lp_ad, l2_lp_Wgi, l2_lp_Wgh, l2_lp_bgi, l2_lp_bgh, l2_pl_Ws, l2_pl_Wd, l2_pl_as, l2_pl_ad, l2_pl_Wgi, l2_pl_Wgh, l2_pl_bgi, l2_pl_bgh, mol_pa_Ws, mol_pa_as, mol_pa_ad, mol_pa_Wgi, mol_pa_Wgh, mol_pa_bgi, mol_pa_bgh, mol_la_Ws, mol_la_as, mol_la_ad, mol_la_Wgi, mol_la_Wgh, mol_la_bgi, mol_la_bgh, lin_pa_W, lin_pa_b, lin_la_W, lin_la_b, mlp_W1, mlp_b1, mlp_W2, mlp_b2):
    ea = {"pp": _pad_cols(edge_attr_pp, 8), "ll": _pad_cols(edge_attr_ll, 8),
          "lp": _pad_cols(edge_attr_lp, 8), "pl": _pad_cols(edge_attr_pl, 8)}
    ei = {"pp": edge_index_pp, "ll": edge_index_ll,
          "lp": edge_index_lp, "pl": edge_index_pl}
    f = x_pa.shape[1]
    fp = 40
    xpa_pad = _pad_cols(x_pa, fp)
    xla_pad = _pad_cols(x_la, fp)

    def l1w(Wg, bg, Wm, bm, Wi0, bi0, Wgi, Wgh, bgi, bgh):
        return (_pad_rows(Wg[:f], fp), _pad_rows(Wg[f:2 * f], fp),
                _pad_rows(Wm[:f], fp), _pad_rows(Wg[2 * f:], 8),
                _pad_rows(Wm[f:], 8), bg, bm, Wi0, bi0, Wgi, Wgh, bgi, bgh)

    p1 = {
        "pp": l1w(l1_pp_Wg, l1_pp_bg, l1_pp_Wm, l1_pp_bm, l1_pp_Wi0, l1_pp_bi0, l1_pp_Wgi, l1_pp_Wgh, l1_pp_bgi, l1_pp_bgh),
        "ll": l1w(l1_ll_Wg, l1_ll_bg, l1_ll_Wm, l1_ll_bm, l1_ll_Wi0, l1_ll_bi0, l1_ll_Wgi, l1_ll_Wgh, l1_ll_bgi, l1_ll_bgh),
        "lp": l1w(l1_lp_Wg, l1_lp_bg, l1_lp_Wm, l1_lp_bm, l1_lp_Wi0, l1_lp_bi0, l1_lp_Wgi, l1_lp_Wgh, l1_lp_bgi, l1_lp_bgh),
        "pl": l1w(l1_pl_Wg, l1_pl_bg, l1_pl_Wm, l1_pl_bm, l1_pl_Wi0, l1_pl_bi0, l1_pl_Wgi, l1_pl_Wgh, l1_pl_bgi, l1_pl_bgh),
    }
    h_pa = _add_relu(
        _gate_conv(xpa_pad, xpa_pad, x_pa, ei["pp"], ea["pp"], *p1["pp"]),
        _gate_conv(xla_pad, xpa_pad, x_pa, ei["lp"], ea["lp"], *p1["lp"]))
    h_la = _add_relu(
        _gate_conv(xla_pad, xla_pad, x_la, ei["ll"], ea["ll"], *p1["ll"]),
        _gate_conv(xpa_pad, xla_pad, x_la, ei["pl"], ea["pl"], *p1["pl"]))

    g_pa = _add_relu(
        _gat_conv(h_pa, h_pa, ei["pp"], l2_pp_Ws, l2_pp_Wd, l2_pp_as, l2_pp_ad, l2_pp_Wgi, l2_pp_Wgh, l2_pp_bgi, l2_pp_bgh),
        _gat_conv(h_la, h_pa, ei["lp"], l2_lp_Ws, l2_lp_Wd, l2_lp_as, l2_lp_ad, l2_lp_Wgi, l2_lp_Wgh, l2_lp_bgi, l2_lp_bgh))
    g_la = _add_relu(
        _gat_conv(h_la, h_la, ei["ll"], l2_ll_Ws, l2_ll_Wd, l2_ll_as, l2_ll_ad, l2_ll_Wgi, l2_ll_Wgh, l2_ll_bgi, l2_ll_bgh),
        _gat_conv(h_pa, h_la, ei["pl"], l2_pl_Ws, l2_pl_Wd, l2_pl_as, l2_pl_ad, l2_pl_Wgi, l2_pl_Wgh, l2_pl_bgi, l2_pl_bgh))

    bpa = batch_pa.astype(jnp.int32)
    bla = batch_la.astype(jnp.int32)

    m_pa = _pool0(g_pa, bpa)
    m_la = _pool0(g_la, bla)

    hs_pa, ls_pa = _proj_attn(g_pa, mol_pa_Ws, mol_pa_as)
    hs_la, ls_la = _proj_attn(g_la, mol_la_Ws, mol_la_as)
    for _ in range(2):
        num, den = _molpool(bpa, ls_pa, hs_pa, m_pa, mol_pa_ad)
        m_pa = _gru(num, m_pa, mol_pa_Wgi, mol_pa_Wgh, mol_pa_bgi, mol_pa_bgh, den=den)
        num, den = _molpool(bla, ls_la, hs_la, m_la, mol_la_ad)
        m_la = _gru(num, m_la, mol_la_Wgi, mol_la_Wgh, mol_la_bgi, mol_la_bgh, den=den)

    return _head(m_pa, m_la, lin_pa_W, lin_pa_b, lin_la_W, lin_la_b,
                 mlp_W1, mlp_b1, mlp_W2, mlp_b2)


# speedup vs baseline: 1.5009x; 1.3802x over previous
import functools

import jax
import jax.numpy as jnp
from jax.experimental import pallas as pl

G_SEG = 256
_SLOPE = 0.2


def _sig(x):
    return 1.0 / (1.0 + jnp.exp(-x))


def _lrelu(x):
    return jnp.where(x > 0, x, _SLOPE * x)


def _blk(n, target):
    b = min(n, target)
    while n % b or b % 8:
        b -= 1
    return b


def _dot(a, b):
    return jnp.dot(a, b, preferred_element_type=jnp.float32,
                   precision=jax.lax.Precision.HIGHEST)


def _dual_mm_kernel(x_ref, w1_ref, b1_ref, w2_ref, b2_ref, o1_ref, o2_ref, *, act2):
    x = x_ref[...]
    o1_ref[...] = _dot(x, w1_ref[...]) + b1_ref[...]
    y = _dot(x, w2_ref[...]) + b2_ref[...]
    o2_ref[...] = _lrelu(y) if act2 else y


def _dual_mm(x, W1, b1, W2, b2, act2):
    n, k = x.shape
    bn = _blk(n, 5000)
    h1 = W1.shape[1]
    h2 = W2.shape[1]
    return pl.pallas_call(
        functools.partial(_dual_mm_kernel, act2=act2),
        grid=(n // bn,),
        in_specs=[
            pl.BlockSpec((bn, k), lambda i: (i, 0)),
            pl.BlockSpec((k, h1), lambda i: (0, 0)),
            pl.BlockSpec((1, h1), lambda i: (0, 0)),
            pl.BlockSpec((k, h2), lambda i: (0, 0)),
            pl.BlockSpec((1, h2), lambda i: (0, 0)),
        ],
        out_specs=[
            pl.BlockSpec((bn, h1), lambda i: (i, 0)),
            pl.BlockSpec((bn, h2), lambda i: (i, 0)),
        ],
        out_shape=[
            jax.ShapeDtypeStruct((n, h1), jnp.float32),
            jax.ShapeDtypeStruct((n, h2), jnp.float32),
        ],
    )(x, W1, b1.reshape(1, -1), W2, b2.reshape(1, -1))


def _proj_attn_kernel(x_ref, w_ref, a_ref, hs_ref, ls_ref, *, fold):
    hs = _dot(x_ref[...], w_ref[...])
    ls = _dot(hs, a_ref[...])
    if fold:
        bn = hs.shape[0]
        hs_ref[...] = jnp.concatenate(
            [hs, ls, jnp.zeros((bn, 7), jnp.float32)], axis=1)
    else:
        hs_ref[...] = hs
        ls_ref[...] = ls


def _proj_attn(x, W, a, fold=False):
    n, k = x.shape
    h = W.shape[1]
    bn = _blk(n, 5000)
    if fold:
        return pl.pallas_call(
            functools.partial(_proj_attn_kernel, ls_ref=None, fold=True),
            grid=(n // bn,),
            in_specs=[
                pl.BlockSpec((bn, k), lambda i: (i, 0)),
                pl.BlockSpec((k, h), lambda i: (0, 0)),
                pl.BlockSpec((h, 1), lambda i: (0, 0)),
            ],
            out_specs=pl.BlockSpec((bn, h + 8), lambda i: (i, 0)),
            out_shape=jax.ShapeDtypeStruct((n, h + 8), jnp.float32),
        )(x, W, a.reshape(-1, 1))
    return pl.pallas_call(
        functools.partial(_proj_attn_kernel, fold=False),
        grid=(n // bn,),
        in_specs=[
            pl.BlockSpec((bn, k), lambda i: (i, 0)),
            pl.BlockSpec((k, h), lambda i: (0, 0)),
            pl.BlockSpec((h, 1), lambda i: (0, 0)),
        ],
        out_specs=[
            pl.BlockSpec((bn, h), lambda i: (i, 0)),
            pl.BlockSpec((bn, 1), lambda i: (i, 0)),
        ],
        out_shape=[
            jax.ShapeDtypeStruct((n, h), jnp.float32),
            jax.ShapeDtypeStruct((n, 1), jnp.float32),
        ],
    )(x, W, a.reshape(-1, 1))


def _attn_scalar_kernel(x_ref, w_ref, a_ref, t_ref):
    t_ref[...] = _dot(_dot(x_ref[...], w_ref[...]), a_ref[...])


def _attn_scalar(x, W, a):
    n, k = x.shape
    h = W.shape[1]
    bn = _blk(n, 5000)
    return pl.pallas_call(
        _attn_scalar_kernel,
        grid=(n // bn,),
        in_specs=[
            pl.BlockSpec((bn, k), lambda i: (i, 0)),
            pl.BlockSpec((k, h), lambda i: (0, 0)),
            pl.BlockSpec((h, 1), lambda i: (0, 0)),
        ],
        out_specs=pl.BlockSpec((bn, 1), lambda i: (i, 0)),
        out_shape=jax.ShapeDtypeStruct((n, 1), jnp.float32),
    )(x, W, a.reshape(-1, 1))


def _edge_l1_kernel(xd_ref, xs_ref, ea_ref, wgd_ref, wgs_ref, wms_ref,
                    wge_ref, wme_ref, bg_ref, bm_ref, msg_ref):
    xd = xd_ref[...]
    xs = xs_ref[...]
    ea = ea_ref[...]
    gate = _sig(_dot(xd, wgd_ref[...]) + _dot(xs, wgs_ref[...]) +
                _dot(ea, wge_ref[...]) + bg_ref[...])
    msg_ref[...] = gate * _lrelu(_dot(xs, wms_ref[...]) +
                                 _dot(ea, wme_ref[...]) + bm_ref[...])


def _edge_l1(xd_g, xs_g, ea_pad, Wgd, Wgs, Wms, WgE, WmE, bg, bm):
    e, kf = xd_g.shape
    ke = ea_pad.shape[1]
    h = Wgd.shape[1]
    be = _blk(e, 4000)
    row40 = pl.BlockSpec((be, kf), lambda i: (i, 0))
    wsp = lambda k: pl.BlockSpec((k, h), lambda i: (0, 0))
    return pl.pallas_call(
        _edge_l1_kernel,
        grid=(e // be,),
        in_specs=[
            row40, row40,
            pl.BlockSpec((be, ke), lambda i: (i, 0)),
            wsp(kf), wsp(kf), wsp(kf), wsp(ke), wsp(ke),
            pl.BlockSpec((1, h), lambda i: (0, 0)),
            pl.BlockSpec((1, h), lambda i: (0, 0)),
        ],
        out_specs=pl.BlockSpec((be, h), lambda i: (i, 0)),
        out_shape=jax.ShapeDtypeStruct((e, h), jnp.float32),
    )(xd_g, xs_g, ea_pad, Wgd, Wgs, Wms, WgE, WmE,
      bg.reshape(1, -1), bm.reshape(1, -1))


def _edge_l2_kernel(g_ref, ld_ref, out_ref):
    g = g_ref[...]
    be = g.shape[0]
    h = g.shape[1] - 8
    ew = jnp.exp(_lrelu(g[:, h:h + 1] + ld_ref[...]))
    out_ref[...] = jnp.concatenate(
        [ew * g[:, :h], ew, jnp.zeros((be, 7), jnp.float32)], axis=1)


def _edge_l2(g_s, ld_d):
    e, h8 = g_s.shape
    be = _blk(e, 4000)
    return pl.pallas_call(
        _edge_l2_kernel,
        grid=(e // be,),
        in_specs=[
            pl.BlockSpec((be, h8), lambda i: (i, 0)),
            pl.BlockSpec((be, 1), lambda i: (i, 0)),
        ],
        out_specs=pl.BlockSpec((be, h8), lambda i: (i, 0)),
        out_shape=jax.ShapeDtypeStruct((e, h8), jnp.float32),
    )(g_s, ld_d)


def _gru_kernel(a_ref, h_ref, wir_ref, wiz_ref, win_ref, whr_ref, whz_ref,
                whn_ref, bir_ref, biz_ref, bin_ref, bhr_ref, bhz_ref,
                bhn_ref, o_ref):
    a = a_ref[...]
    h = h_ref[...]
    r = _sig(_dot(a, wir_ref[...]) + bir_ref[...] + _dot(h, whr_ref[...]) + bhr_ref[...])
    z = _sig(_dot(a, wiz_ref[...]) + biz_ref[...] + _dot(h, whz_ref[...]) + bhz_ref[...])
    nn = jnp.tanh(_dot(a, win_ref[...]) + bin_ref[...] + r * (_dot(h, whn_ref[...]) + bhn_ref[...]))
    o_ref[...] = (1.0 - z) * nn + z * h


def _gru_div_kernel(num_ref, den_ref, h_ref, wir_ref, wiz_ref, win_ref,
                    whr_ref, whz_ref, whn_ref, bir_ref, biz_ref, bin_ref,
                    bhr_ref, bhz_ref, bhn_ref, o_ref):
    a = num_ref[...] / (den_ref[...] + 1e-16)
    h = h_ref[...]
    r = _sig(_dot(a, wir_ref[...]) + bir_ref[...] + _dot(h, whr_ref[...]) + bhr_ref[...])
    z = _sig(_dot(a, wiz_ref[...]) + biz_ref[...] + _dot(h, whz_ref[...]) + bhz_ref[...])
    nn = jnp.tanh(_dot(a, win_ref[...]) + bin_ref[...] + r * (_dot(h, whn_ref[...]) + bhn_ref[...]))
    o_ref[...] = (1.0 - z) * nn + z * h


def _split_gru_w(Wgi, Wgh, bgi, bgh):
    h = Wgi.shape[0]
    ws = [Wgi[:, :h], Wgi[:, h:2 * h], Wgi[:, 2 * h:],
          Wgh[:, :h], Wgh[:, h:2 * h], Wgh[:, 2 * h:]]
    bs = [bgi[:h], bgi[h:2 * h], bgi[2 * h:],
          bgh[:h], bgh[h:2 * h], bgh[2 * h:]]
    return ws, [b.reshape(1, -1) for b in bs]


def _gru(a, h, Wgi, Wgh, bgi, bgh, den=None):
    n, hh = a.shape
    bn = _blk(n, 5000)
    ws, bs = _split_gru_w(Wgi, Wgh, bgi, bgh)
    wspec = [pl.BlockSpec((hh, hh), lambda i: (0, 0))] * 6
    bspec = [pl.BlockSpec((1, hh), lambda i: (0, 0))] * 6
    row = pl.BlockSpec((bn, hh), lambda i: (i, 0))
    col = pl.BlockSpec((bn, 1), lambda i: (i, 0))
    if den is None:
        return pl.pallas_call(
            _gru_kernel,
            grid=(n // bn,),
            in_specs=[row, row] + wspec + bspec,
            out_specs=row,
            out_shape=jax.ShapeDtypeStruct((n, hh), jnp.float32),
        )(a, h, *ws, *bs)
    return pl.pallas_call(
        _gru_div_kernel,
        grid=(n // bn,),
        in_specs=[row, col, row] + wspec + bspec,
        out_specs=row,
        out_shape=jax.ShapeDtypeStruct((n, hh), jnp.float32),
    )(a, den, h, *ws, *bs)


def _add_relu_kernel(a_ref, b_ref, o_ref):
    o_ref[...] = jnp.maximum(a_ref[...] + b_ref[...], 0.0)


def _add_relu(a, b):
    n, h = a.shape
    bn = _blk(n, 5000)
    row = pl.BlockSpec((bn, h), lambda i: (i, 0))
    return pl.pallas_call(
        _add_relu_kernel,
        grid=(n // bn,),
        in_specs=[row, row],
        out_specs=row,
        out_shape=jax.ShapeDtypeStruct((n, h), jnp.float32),
    )(a, b)


def _pool0_kernel(g_ref, brow_ref, o_ref):
    i = pl.program_id(0)

    @pl.when(i == 0)
    def _():
        o_ref[...] = jnp.zeros_like(o_ref)

    bn = g_ref.shape[0]
    brow = brow_ref[...].reshape(1, bn)
    oht = jnp.where(
        jax.lax.broadcasted_iota(jnp.int32, (G_SEG, bn), 0) == brow,
        1.0, 0.0)
    o_ref[...] += _dot(oht, g_ref[...])

    @pl.when(i == pl.num_programs(0) - 1)
    def _():
        o_ref[...] = jnp.maximum(o_ref[...], 0.0)


def _pool0(g, batch):
    n, h = g.shape
    bn = _blk(n, 5000)
    batch_row = batch.reshape(n // bn, 1, bn)
    return pl.pallas_call(
        _pool0_kernel,
        grid=(n // bn,),
        in_specs=[
            pl.BlockSpec((bn, h), lambda i: (i, 0)),
            pl.BlockSpec((1, 1, bn), lambda i: (i, 0, 0)),
        ],
        out_specs=pl.BlockSpec((G_SEG, h), lambda i: (0, 0)),
        out_shape=jax.ShapeDtypeStruct((G_SEG, h), jnp.float32),
    )(g, batch_row)


def _molpool_kernel(bcol_ref, brow_ref, ls_ref, hs_ref, xmol_ref, ad_ref,
                    num_ref, den_ref):
    i = pl.program_id(0)

    @pl.when(i == 0)
    def _():
        num_ref[...] = jnp.zeros_like(num_ref)
        den_ref[...] = jnp.zeros_like(den_ref)

    bn = hs_ref.shape[0]
    t = _dot(xmol_ref[...], ad_ref[...])
    ohg = jnp.where(
        jax.lax.broadcasted_iota(jnp.int32, (bn, G_SEG), 1) == bcol_ref[...],
        1.0, 0.0)
    td = _dot(ohg, t)
    e = jnp.exp(_lrelu(ls_ref[...] + td))
    brow = brow_ref[...].reshape(1, bn)
    oht = jnp.where(
        jax.lax.broadcasted_iota(jnp.int32, (G_SEG, bn), 0) == brow,
        1.0, 0.0)
    num_ref[...] += _dot(oht, e * hs_ref[...])
    den_ref[...] += _dot(oht, e)


def _molpool(batch, ls, hs, x_mol, ad):
    n, h = hs.shape
    bn = _blk(n, 5000)
    batch_col = batch.reshape(n, 1)
    batch_row = batch.reshape(n // bn, 1, bn)
    return pl.pallas_call(
        _molpool_kernel,
        grid=(n // bn,),
        in_specs=[
            pl.BlockSpec((bn, 1), lambda i: (i, 0)),
            pl.BlockSpec((1, 1, bn), lambda i: (i, 0, 0)),
            pl.BlockSpec((bn, 1), lambda i: (i, 0)),
            pl.BlockSpec((bn, h), lambda i: (i, 0)),
            pl.BlockSpec((G_SEG, h), lambda i: (0, 0)),
            pl.BlockSpec((h, 1), lambda i: (0, 0)),
        ],
        out_specs=[
            pl.BlockSpec((G_SEG, h), lambda i: (0, 0)),
            pl.BlockSpec((G_SEG, 1), lambda i: (0, 0)),
        ],
        out_shape=[
            jax.ShapeDtypeStruct((G_SEG, h), jnp.float32),
            jax.ShapeDtypeStruct((G_SEG, 1), jnp.float32),
        ],
    )(batch_col, batch_row, ls, hs, x_mol, ad.reshape(-1, 1))


def _head_kernel(mpa_ref, mla_ref, wpa_ref, bpa_ref, wla_ref, bla_ref,
                 w1a_ref, w1b_ref, b1_ref, w2_ref, b2_ref, y_ref):
    y_pa = _dot(mpa_ref[...], wpa_ref[...]) + bpa_ref[...]
    y_la = _dot(mla_ref[...], wla_ref[...]) + bla_ref[...]
    z = jnp.maximum(_dot(y_pa, w1a_ref[...]) + _dot(y_la, w1b_ref[...]) + b1_ref[...], 0.0)
    y_ref[...] = _dot(z, w2_ref[...]) + b2_ref[...]


def _head(m_pa, m_la, lin_pa_W, lin_pa_b, lin_la_W, lin_la_b, mlp_W1, mlp_b1,
          mlp_W2, mlp_b2):
    h = m_pa.shape[1]
    return pl.pallas_call(
        _head_kernel,
        out_shape=jax.ShapeDtypeStruct((G_SEG, 1), jnp.float32),
    )(m_pa, m_la, lin_pa_W, lin_pa_b.reshape(1, -1), lin_la_W,
      lin_la_b.reshape(1, -1), mlp_W1[:h], mlp_W1[h:], mlp_b1.reshape(1, -1),
      mlp_W2, mlp_b2.reshape(1, -1))


def _mm_act_kernel(x_ref, w_ref, b_ref, o_ref):
    o_ref[...] = _lrelu(_dot(x_ref[...], w_ref[...]) + b_ref[...])


def _mm_act(x, W, b):
    n, k = x.shape
    h = W.shape[1]
    bn = _blk(n, 5000)
    return pl.pallas_call(
        _mm_act_kernel,
        grid=(n // bn,),
        in_specs=[
            pl.BlockSpec((bn, k), lambda i: (i, 0)),
            pl.BlockSpec((k, h), lambda i: (0, 0)),
            pl.BlockSpec((1, h), lambda i: (0, 0)),
        ],
        out_specs=pl.BlockSpec((bn, h), lambda i: (i, 0)),
        out_shape=jax.ShapeDtypeStruct((n, h), jnp.float32),
    )(x, W, b.reshape(1, -1))


def _gate_conv(xs_pad, xd_pad, xd, ei, ea_pad, Wgd, Wgs, Wms, WgE, WmE,
               bg, bm, Wi0, bi0, Wgi, Wgh, bgi, bgh):
    s, d = ei[0], ei[1]
    h0 = _mm_act(xd, Wi0, bi0)
    msg = _edge_l1(xd_pad[d], xs_pad[s], ea_pad, Wgd, Wgs, Wms, WgE, WmE, bg, bm)
    agg = jax.ops.segment_sum(msg, d, num_segments=xd.shape[0])
    return _gru(agg, h0, Wgi, Wgh, bgi, bgh)


def _gat_conv(xs, xd, ei, Ws, Wd, a_s, a_d, Wgi, Wgh, bgi, bgh):
    s, d = ei[0], ei[1]
    h = xd.shape[1]
    hls = _proj_attn(xs, Ws, a_s, fold=True)
    ld = _attn_scalar(xd, Wd, a_d)
    comb = _edge_l2(hls[s], ld[d])
    seg = jax.ops.segment_sum(comb, d, num_segments=xd.shape[0])
    return _gru(seg[:, :h], xd, Wgi, Wgh, bgi, bgh, den=seg[:, h:h + 1])


def _pad_cols(x, to):
    n, f = x.shape
    return jnp.concatenate([x, jnp.zeros((n, to - f), jnp.float32)], axis=1)


def _pad_rows(W, to):
    f, h = W.shape
    return jnp.concatenate([W, jnp.zeros((to - f, h), jnp.float32)], axis=0)


def kernel(x_pa, x_la, edge_index_pp, edge_attr_pp, edge_index_ll, edge_attr_ll, edge_index_lp, edge_attr_lp, edge_index_pl, edge_attr_pl, batch_pa, batch_la, l1_pp_Wg, l1_pp_bg, l1_pp_Wm, l1_pp_bm, l1_pp_Wi0, l1_pp_bi0, l1_pp_Wgi, l1_pp_Wgh, l1_pp_bgi, l1_pp_bgh, l1_ll_Wg, l1_ll_bg, l1_ll_Wm, l1_ll_bm, l1_ll_Wi0, l1_ll_bi0, l1_ll_Wgi, l1_ll_Wgh, l1_ll_bgi, l1_ll_bgh, l1_lp_Wg, l1_lp_bg, l1_lp_Wm, l1_lp_bm, l1_lp_Wi0, l1_lp_bi0, l1_lp_Wgi, l1_lp_Wgh, l1_lp_bgi, l1_lp_bgh, l1_pl_Wg, l1_pl_bg, l1_pl_Wm, l1_pl_bm, l1_pl_Wi0, l1_pl_bi0, l1_pl_Wgi, l1_pl_Wgh, l1_pl_bgi, l1_pl_bgh, l2_pp_Ws, l2_pp_Wd, l2_pp_as, l2_pp_ad, l2_pp_Wgi, l2_pp_Wgh, l2_pp_bgi, l2_pp_bgh, l2_ll_Ws, l2_ll_Wd, l2_ll_as, l2_ll_ad, l2_ll_Wgi, l2_ll_Wgh, l2_ll_bgi, l2_ll_bgh, l2_lp_Ws, l2_lp_Wd, l2_lp_as, l2_lp_ad, l2_lp_Wgi, l2_lp_Wgh, l2_lp_bgi, l2_lp_bgh, l2_pl_Ws, l2_pl_Wd, l2_pl_as, l2_pl_ad, l2_pl_Wgi, l2_pl_Wgh, l2_pl_bgi, l2_pl_bgh, mol_pa_Ws, mol_pa_as, mol_pa_ad, mol_pa_Wgi, mol_pa_Wgh, mol_pa_bgi, mol_pa_bgh, mol_la_Ws, mol_la_as, mol_la_ad, mol_la_Wgi, mol_la_Wgh, mol_la_bgi, mol_la_bgh, lin_pa_W, lin_pa_b, lin_la_W, lin_la_b, mlp_W1, mlp_b1, mlp_W2, mlp_b2):
    ea = {"pp": _pad_cols(edge_attr_pp, 8), "ll": _pad_cols(edge_attr_ll, 8),
          "lp": _pad_cols(edge_attr_lp, 8), "pl": _pad_cols(edge_attr_pl, 8)}
    ei = {"pp": edge_index_pp, "ll": edge_index_ll,
          "lp": edge_index_lp, "pl": edge_index_pl}
    f = x_pa.shape[1]
    fp = 40
    xpa_pad = _pad_cols(x_pa, fp)
    xla_pad = _pad_cols(x_la, fp)

    def l1w(Wg, bg, Wm, bm, Wi0, bi0, Wgi, Wgh, bgi, bgh):
        return (_pad_rows(Wg[:f], fp), _pad_rows(Wg[f:2 * f], fp),
                _pad_rows(Wm[:f], fp), _pad_rows(Wg[2 * f:], 8),
                _pad_rows(Wm[f:], 8), bg, bm, Wi0, bi0, Wgi, Wgh, bgi, bgh)

    p1 = {
        "pp": l1w(l1_pp_Wg, l1_pp_bg, l1_pp_Wm, l1_pp_bm, l1_pp_Wi0, l1_pp_bi0, l1_pp_Wgi, l1_pp_Wgh, l1_pp_bgi, l1_pp_bgh),
        "ll": l1w(l1_ll_Wg, l1_ll_bg, l1_ll_Wm, l1_ll_bm, l1_ll_Wi0, l1_ll_bi0, l1_ll_Wgi, l1_ll_Wgh, l1_ll_bgi, l1_ll_bgh),
        "lp": l1w(l1_lp_Wg, l1_lp_bg, l1_lp_Wm, l1_lp_bm, l1_lp_Wi0, l1_lp_bi0, l1_lp_Wgi, l1_lp_Wgh, l1_lp_bgi, l1_lp_bgh),
        "pl": l1w(l1_pl_Wg, l1_pl_bg, l1_pl_Wm, l1_pl_bm, l1_pl_Wi0, l1_pl_bi0, l1_pl_Wgi, l1_pl_Wgh, l1_pl_bgi, l1_pl_bgh),
    }
    h_pa = _add_relu(
        _gate_conv(xpa_pad, xpa_pad, x_pa, ei["pp"], ea["pp"], *p1["pp"]),
        _gate_conv(xla_pad, xpa_pad, x_pa, ei["lp"], ea["lp"], *p1["lp"]))
    h_la = _add_relu(
        _gate_conv(xla_pad, xla_pad, x_la, ei["ll"], ea["ll"], *p1["ll"]),
        _gate_conv(xpa_pad, xla_pad, x_la, ei["pl"], ea["pl"], *p1["pl"]))

    g_pa = _add_relu(
        _gat_conv(h_pa, h_pa, ei["pp"], l2_pp_Ws, l2_pp_Wd, l2_pp_as, l2_pp_ad, l2_pp_Wgi, l2_pp_Wgh, l2_pp_bgi, l2_pp_bgh),
        _gat_conv(h_la, h_pa, ei["lp"], l2_lp_Ws, l2_lp_Wd, l2_lp_as, l2_lp_ad, l2_lp_Wgi, l2_lp_Wgh, l2_lp_bgi, l2_lp_bgh))
    g_la = _add_relu(
        _gat_conv(h_la, h_la, ei["ll"], l2_ll_Ws, l2_ll_Wd, l2_ll_as, l2_ll_ad, l2_ll_Wgi, l2_ll_Wgh, l2_ll_bgi, l2_ll_bgh),
        _gat_conv(h_pa, h_la, ei["pl"], l2_pl_Ws, l2_pl_Wd, l2_pl_as, l2_pl_ad, l2_pl_Wgi, l2_pl_Wgh, l2_pl_bgi, l2_pl_bgh))

    bpa = batch_pa.astype(jnp.int32)
    bla = batch_la.astype(jnp.int32)

    m_pa = _pool0(g_pa, bpa)
    m_la = _pool0(g_la, bla)

    hs_pa, ls_pa = _proj_attn(g_pa, mol_pa_Ws, mol_pa_as)
    hs_la, ls_la = _proj_attn(g_la, mol_la_Ws, mol_la_as)
    for _ in range(2):
        num, den = _molpool(bpa, ls_pa, hs_pa, m_pa, mol_pa_ad)
        m_pa = _gru(num, m_pa, mol_pa_Wgi, mol_pa_Wgh, mol_pa_bgi, mol_pa_bgh, den=den)
        num, den = _molpool(bla, ls_la, hs_la, m_la, mol_la_ad)
        m_la = _gru(num, m_la, mol_la_Wgi, mol_la_Wgh, mol_la_bgi, mol_la_bgh, den=den)

    return _head(m_pa, m_la, lin_pa_W, lin_pa_b, lin_la_W, lin_la_b,
                 mlp_W1, mlp_b1, mlp_W2, mlp_b2)


# speedup vs baseline: 2.1137x; 1.4083x over previous
import functools

import jax
import jax.numpy as jnp
from jax.experimental import pallas as pl

G_SEG = 256
_SLOPE = 0.2


def _sig(x):
    return 1.0 / (1.0 + jnp.exp(-x))


def _lrelu(x):
    return jnp.where(x > 0, x, _SLOPE * x)


def _blk(n, target):
    b = min(n, target)
    while n % b or b % 8:
        b -= 1
    return b


def _dot(a, b):
    return jnp.dot(a, b, preferred_element_type=jnp.float32,
                   precision=jax.lax.Precision.HIGHEST)


def _dual_mm_kernel(x_ref, w1_ref, b1_ref, w2_ref, b2_ref, o1_ref, o2_ref, *, act2):
    x = x_ref[...]
    o1_ref[...] = _dot(x, w1_ref[...]) + b1_ref[...]
    y = _dot(x, w2_ref[...]) + b2_ref[...]
    o2_ref[...] = _lrelu(y) if act2 else y


def _dual_mm(x, W1, b1, W2, b2, act2):
    n, k = x.shape
    bn = _blk(n, 5000)
    h1 = W1.shape[1]
    h2 = W2.shape[1]
    return pl.pallas_call(
        functools.partial(_dual_mm_kernel, act2=act2),
        grid=(n // bn,),
        in_specs=[
            pl.BlockSpec((bn, k), lambda i: (i, 0)),
            pl.BlockSpec((k, h1), lambda i: (0, 0)),
            pl.BlockSpec((1, h1), lambda i: (0, 0)),
            pl.BlockSpec((k, h2), lambda i: (0, 0)),
            pl.BlockSpec((1, h2), lambda i: (0, 0)),
        ],
        out_specs=[
            pl.BlockSpec((bn, h1), lambda i: (i, 0)),
            pl.BlockSpec((bn, h2), lambda i: (i, 0)),
        ],
        out_shape=[
            jax.ShapeDtypeStruct((n, h1), jnp.float32),
            jax.ShapeDtypeStruct((n, h2), jnp.float32),
        ],
    )(x, W1, b1.reshape(1, -1), W2, b2.reshape(1, -1))


def _proj_attn_kernel(x_ref, w_ref, a_ref, hs_ref, ls_ref, *, fold):
    hs = _dot(x_ref[...], w_ref[...])
    ls = _dot(hs, a_ref[...])
    if fold:
        bn = hs.shape[0]
        hs_ref[...] = jnp.concatenate(
            [hs, ls, jnp.zeros((bn, 7), jnp.float32)], axis=1)
    else:
        hs_ref[...] = hs
        ls_ref[...] = ls


def _proj_attn(x, W, a, fold=False):
    n, k = x.shape
    h = W.shape[1]
    bn = _blk(n, 5000)
    if fold:
        return pl.pallas_call(
            functools.partial(_proj_attn_kernel, ls_ref=None, fold=True),
            grid=(n // bn,),
            in_specs=[
                pl.BlockSpec((bn, k), lambda i: (i, 0)),
                pl.BlockSpec((k, h), lambda i: (0, 0)),
                pl.BlockSpec((h, 1), lambda i: (0, 0)),
            ],
            out_specs=pl.BlockSpec((bn, h + 8), lambda i: (i, 0)),
            out_shape=jax.ShapeDtypeStruct((n, h + 8), jnp.float32),
        )(x, W, a.reshape(-1, 1))
    return pl.pallas_call(
        functools.partial(_proj_attn_kernel, fold=False),
        grid=(n // bn,),
        in_specs=[
            pl.BlockSpec((bn, k), lambda i: (i, 0)),
            pl.BlockSpec((k, h), lambda i: (0, 0)),
            pl.BlockSpec((h, 1), lambda i: (0, 0)),
        ],
        out_specs=[
            pl.BlockSpec((bn, h), lambda i: (i, 0)),
            pl.BlockSpec((bn, 1), lambda i: (i, 0)),
        ],
        out_shape=[
            jax.ShapeDtypeStruct((n, h), jnp.float32),
            jax.ShapeDtypeStruct((n, 1), jnp.float32),
        ],
    )(x, W, a.reshape(-1, 1))


def _attn_scalar_kernel(x_ref, w_ref, a_ref, t_ref):
    t = _dot(_dot(x_ref[...], w_ref[...]), a_ref[...])
    bn = t.shape[0]
    t_ref[...] = jnp.concatenate([t, jnp.zeros((bn, 7), jnp.float32)], axis=1)


def _attn_scalar(x, W, a):
    n, k = x.shape
    h = W.shape[1]
    bn = _blk(n, 5000)
    return pl.pallas_call(
        _attn_scalar_kernel,
        grid=(n // bn,),
        in_specs=[
            pl.BlockSpec((bn, k), lambda i: (i, 0)),
            pl.BlockSpec((k, h), lambda i: (0, 0)),
            pl.BlockSpec((h, 1), lambda i: (0, 0)),
        ],
        out_specs=pl.BlockSpec((bn, 8), lambda i: (i, 0)),
        out_shape=jax.ShapeDtypeStruct((n, 8), jnp.float32),
    )(x, W, a.reshape(-1, 1))


def _edge_l1_kernel(xd_ref, xs_ref, ea_ref, wgd_ref, wgs_ref, wms_ref,
                    wge_ref, wme_ref, bg_ref, bm_ref, msg_ref):
    xd = xd_ref[...]
    xs = xs_ref[...]
    ea = ea_ref[...]
    gate = _sig(_dot(xd, wgd_ref[...]) + _dot(xs, wgs_ref[...]) +
                _dot(ea, wge_ref[...]) + bg_ref[...])
    msg_ref[...] = gate * _lrelu(_dot(xs, wms_ref[...]) +
                                 _dot(ea, wme_ref[...]) + bm_ref[...])


def _edge_l1(xd_g, xs_g, ea_pad, Wgd, Wgs, Wms, WgE, WmE, bg, bm):
    e, kf = xd_g.shape
    ke = ea_pad.shape[1]
    h = Wgd.shape[1]
    be = _blk(e, 4000)
    row40 = pl.BlockSpec((be, kf), lambda i: (i, 0))
    wsp = lambda k: pl.BlockSpec((k, h), lambda i: (0, 0))
    return pl.pallas_call(
        _edge_l1_kernel,
        grid=(e // be,),
        in_specs=[
            row40, row40,
            pl.BlockSpec((be, ke), lambda i: (i, 0)),
            wsp(kf), wsp(kf), wsp(kf), wsp(ke), wsp(ke),
            pl.BlockSpec((1, h), lambda i: (0, 0)),
            pl.BlockSpec((1, h), lambda i: (0, 0)),
        ],
        out_specs=pl.BlockSpec((be, h), lambda i: (i, 0)),
        out_shape=jax.ShapeDtypeStruct((e, h), jnp.float32),
    )(xd_g, xs_g, ea_pad, Wgd, Wgs, Wms, WgE, WmE,
      bg.reshape(1, -1), bm.reshape(1, -1))


def _edge_l2_kernel(g_ref, ld_ref, out_ref):
    g = g_ref[...]
    be = g.shape[0]
    h = g.shape[1] - 8
    ew = jnp.exp(_lrelu(g[:, h:h + 1] + ld_ref[...][:, 0:1]))
    out_ref[...] = jnp.concatenate(
        [ew * g[:, :h], ew, jnp.zeros((be, 7), jnp.float32)], axis=1)


def _edge_l2(g_s, ld_d):
    e, h8 = g_s.shape
    be = _blk(e, 4000)
    return pl.pallas_call(
        _edge_l2_kernel,
        grid=(e // be,),
        in_specs=[
            pl.BlockSpec((be, h8), lambda i: (i, 0)),
            pl.BlockSpec((be, 8), lambda i: (i, 0)),
        ],
        out_specs=pl.BlockSpec((be, h8), lambda i: (i, 0)),
        out_shape=jax.ShapeDtypeStruct((e, h8), jnp.float32),
    )(g_s, ld_d)


def _gru_kernel(a_ref, h_ref, wir_ref, wiz_ref, win_ref, whr_ref, whz_ref,
                whn_ref, bir_ref, biz_ref, bin_ref, bhr_ref, bhz_ref,
                bhn_ref, o_ref):
    a = a_ref[...]
    h = h_ref[...]
    r = _sig(_dot(a, wir_ref[...]) + bir_ref[...] + _dot(h, whr_ref[...]) + bhr_ref[...])
    z = _sig(_dot(a, wiz_ref[...]) + biz_ref[...] + _dot(h, whz_ref[...]) + bhz_ref[...])
    nn = jnp.tanh(_dot(a, win_ref[...]) + bin_ref[...] + r * (_dot(h, whn_ref[...]) + bhn_ref[...]))
    o_ref[...] = (1.0 - z) * nn + z * h


def _gru_div_kernel(num_ref, den_ref, h_ref, wir_ref, wiz_ref, win_ref,
                    whr_ref, whz_ref, whn_ref, bir_ref, biz_ref, bin_ref,
                    bhr_ref, bhz_ref, bhn_ref, o_ref):
    a = num_ref[...] / (den_ref[...] + 1e-16)
    h = h_ref[...]
    r = _sig(_dot(a, wir_ref[...]) + bir_ref[...] + _dot(h, whr_ref[...]) + bhr_ref[...])
    z = _sig(_dot(a, wiz_ref[...]) + biz_ref[...] + _dot(h, whz_ref[...]) + bhz_ref[...])
    nn = jnp.tanh(_dot(a, win_ref[...]) + bin_ref[...] + r * (_dot(h, whn_ref[...]) + bhn_ref[...]))
    o_ref[...] = (1.0 - z) * nn + z * h


def _split_gru_w(Wgi, Wgh, bgi, bgh):
    h = Wgi.shape[0]
    ws = [Wgi[:, :h], Wgi[:, h:2 * h], Wgi[:, 2 * h:],
          Wgh[:, :h], Wgh[:, h:2 * h], Wgh[:, 2 * h:]]
    bs = [bgi[:h], bgi[h:2 * h], bgi[2 * h:],
          bgh[:h], bgh[h:2 * h], bgh[2 * h:]]
    return ws, [b.reshape(1, -1) for b in bs]


def _gru(a, h, Wgi, Wgh, bgi, bgh, den=None):
    n, hh = a.shape
    bn = _blk(n, 5000)
    ws, bs = _split_gru_w(Wgi, Wgh, bgi, bgh)
    wspec = [pl.BlockSpec((hh, hh), lambda i: (0, 0))] * 6
    bspec = [pl.BlockSpec((1, hh), lambda i: (0, 0))] * 6
    row = pl.BlockSpec((bn, hh), lambda i: (i, 0))
    col = pl.BlockSpec((bn, 1), lambda i: (i, 0))
    if den is None:
        return pl.pallas_call(
            _gru_kernel,
            grid=(n // bn,),
            in_specs=[row, row] + wspec + bspec,
            out_specs=row,
            out_shape=jax.ShapeDtypeStruct((n, hh), jnp.float32),
        )(a, h, *ws, *bs)
    return pl.pallas_call(
        _gru_div_kernel,
        grid=(n // bn,),
        in_specs=[row, col, row] + wspec + bspec,
        out_specs=row,
        out_shape=jax.ShapeDtypeStruct((n, hh), jnp.float32),
    )(a, den, h, *ws, *bs)


def _add_relu_kernel(a_ref, b_ref, o_ref):
    o_ref[...] = jnp.maximum(a_ref[...] + b_ref[...], 0.0)


def _add_relu(a, b):
    n, h = a.shape
    bn = _blk(n, 5000)
    row = pl.BlockSpec((bn, h), lambda i: (i, 0))
    return pl.pallas_call(
        _add_relu_kernel,
        grid=(n // bn,),
        in_specs=[row, row],
        out_specs=row,
        out_shape=jax.ShapeDtypeStruct((n, h), jnp.float32),
    )(a, b)


def _pool0_kernel(g_ref, brow_ref, o_ref):
    i = pl.program_id(0)

    @pl.when(i == 0)
    def _():
        o_ref[...] = jnp.zeros_like(o_ref)

    bn = g_ref.shape[0]
    brow = brow_ref[...].reshape(1, bn)
    oht = jnp.where(
        jax.lax.broadcasted_iota(jnp.int32, (G_SEG, bn), 0) == brow,
        1.0, 0.0)
    o_ref[...] += _dot(oht, g_ref[...])

    @pl.when(i == pl.num_programs(0) - 1)
    def _():
        o_ref[...] = jnp.maximum(o_ref[...], 0.0)


def _pool0(g, batch):
    n, h = g.shape
    bn = _blk(n, 5000)
    batch_row = batch.reshape(n // bn, 1, bn)
    return pl.pallas_call(
        _pool0_kernel,
        grid=(n // bn,),
        in_specs=[
            pl.BlockSpec((bn, h), lambda i: (i, 0)),
            pl.BlockSpec((1, 1, bn), lambda i: (i, 0, 0)),
        ],
        out_specs=pl.BlockSpec((G_SEG, h), lambda i: (0, 0)),
        out_shape=jax.ShapeDtypeStruct((G_SEG, h), jnp.float32),
    )(g, batch_row)


def _molpool_kernel(bcol_ref, brow_ref, ls_ref, hs_ref, xmol_ref, ad_ref,
                    num_ref, den_ref):
    i = pl.program_id(0)

    @pl.when(i == 0)
    def _():
        num_ref[...] = jnp.zeros_like(num_ref)
        den_ref[...] = jnp.zeros_like(den_ref)

    bn = hs_ref.shape[0]
    t = _dot(xmol_ref[...], ad_ref[...])
    ohg = jnp.where(
        jax.lax.broadcasted_iota(jnp.int32, (bn, G_SEG), 1) == bcol_ref[...],
        1.0, 0.0)
    td = _dot(ohg, t)
    e = jnp.exp(_lrelu(ls_ref[...] + td))
    brow = brow_ref[...].reshape(1, bn)
    oht = jnp.where(
        jax.lax.broadcasted_iota(jnp.int32, (G_SEG, bn), 0) == brow,
        1.0, 0.0)
    num_ref[...] += _dot(oht, e * hs_ref[...])
    den_ref[...] += _dot(oht, e)


def _molpool(batch, ls, hs, x_mol, ad):
    n, h = hs.shape
    bn = _blk(n, 5000)
    batch_col = batch.reshape(n, 1)
    batch_row = batch.reshape(n // bn, 1, bn)
    return pl.pallas_call(
        _molpool_kernel,
        grid=(n // bn,),
        in_specs=[
            pl.BlockSpec((bn, 1), lambda i: (i, 0)),
            pl.BlockSpec((1, 1, bn), lambda i: (i, 0, 0)),
            pl.BlockSpec((bn, 1), lambda i: (i, 0)),
            pl.BlockSpec((bn, h), lambda i: (i, 0)),
            pl.BlockSpec((G_SEG, h), lambda i: (0, 0)),
            pl.BlockSpec((h, 1), lambda i: (0, 0)),
        ],
        out_specs=[
            pl.BlockSpec((G_SEG, h), lambda i: (0, 0)),
            pl.BlockSpec((G_SEG, 1), lambda i: (0, 0)),
        ],
        out_shape=[
            jax.ShapeDtypeStruct((G_SEG, h), jnp.float32),
            jax.ShapeDtypeStruct((G_SEG, 1), jnp.float32),
        ],
    )(batch_col, batch_row, ls, hs, x_mol, ad.reshape(-1, 1))


def _head_kernel(mpa_ref, mla_ref, wpa_ref, bpa_ref, wla_ref, bla_ref,
                 w1a_ref, w1b_ref, b1_ref, w2_ref, b2_ref, y_ref):
    y_pa = _dot(mpa_ref[...], wpa_ref[...]) + bpa_ref[...]
    y_la = _dot(mla_ref[...], wla_ref[...]) + bla_ref[...]
    z = jnp.maximum(_dot(y_pa, w1a_ref[...]) + _dot(y_la, w1b_ref[...]) + b1_ref[...], 0.0)
    y_ref[...] = _dot(z, w2_ref[...]) + b2_ref[...]


def _head(m_pa, m_la, lin_pa_W, lin_pa_b, lin_la_W, lin_la_b, mlp_W1, mlp_b1,
          mlp_W2, mlp_b2):
    h = m_pa.shape[1]
    return pl.pallas_call(
        _head_kernel,
        out_shape=jax.ShapeDtypeStruct((G_SEG, 1), jnp.float32),
    )(m_pa, m_la, lin_pa_W, lin_pa_b.reshape(1, -1), lin_la_W,
      lin_la_b.reshape(1, -1), mlp_W1[:h], mlp_W1[h:], mlp_b1.reshape(1, -1),
      mlp_W2, mlp_b2.reshape(1, -1))


def _mm_act_kernel(x_ref, w_ref, b_ref, o_ref):
    o_ref[...] = _lrelu(_dot(x_ref[...], w_ref[...]) + b_ref[...])


def _mm_act(x, W, b):
    n, k = x.shape
    h = W.shape[1]
    bn = _blk(n, 5000)
    return pl.pallas_call(
        _mm_act_kernel,
        grid=(n // bn,),
        in_specs=[
            pl.BlockSpec((bn, k), lambda i: (i, 0)),
            pl.BlockSpec((k, h), lambda i: (0, 0)),
            pl.BlockSpec((1, h), lambda i: (0, 0)),
        ],
        out_specs=pl.BlockSpec((bn, h), lambda i: (i, 0)),
        out_shape=jax.ShapeDtypeStruct((n, h), jnp.float32),
    )(x, W, b.reshape(1, -1))


def _gate_conv(xs_pad, xd_pad, xd, ei, ea_pad, Wgd, Wgs, Wms, WgE, WmE,
               bg, bm, Wi0, bi0, Wgi, Wgh, bgi, bgh):
    s, d = ei[0], ei[1]
    h0 = _mm_act(xd, Wi0, bi0)
    msg = _edge_l1(xd_pad[d], xs_pad[s], ea_pad, Wgd, Wgs, Wms, WgE, WmE, bg, bm)
    agg = jax.ops.segment_sum(msg, d, num_segments=xd.shape[0])
    return _gru(agg, h0, Wgi, Wgh, bgi, bgh)


def _gat_conv(xs, xd, ei, Ws, Wd, a_s, a_d, Wgi, Wgh, bgi, bgh):
    s, d = ei[0], ei[1]
    h = xd.shape[1]
    hls = _proj_attn(xs, Ws, a_s, fold=True)
    ld = _attn_scalar(xd, Wd, a_d)
    comb = _edge_l2(hls[s], ld[d])
    seg = jax.ops.segment_sum(comb, d, num_segments=xd.shape[0])
    return _gru(seg[:, :h], xd, Wgi, Wgh, bgi, bgh, den=seg[:, h:h + 1])


def _pad_cols(x, to):
    n, f = x.shape
    return jnp.concatenate([x, jnp.zeros((n, to - f), jnp.float32)], axis=1)


def _pad_rows(W, to):
    f, h = W.shape
    return jnp.concatenate([W, jnp.zeros((to - f, h), jnp.float32)], axis=0)


def kernel(x_pa, x_la, edge_index_pp, edge_attr_pp, edge_index_ll, edge_attr_ll, edge_index_lp, edge_attr_lp, edge_index_pl, edge_attr_pl, batch_pa, batch_la, l1_pp_Wg, l1_pp_bg, l1_pp_Wm, l1_pp_bm, l1_pp_Wi0, l1_pp_bi0, l1_pp_Wgi, l1_pp_Wgh, l1_pp_bgi, l1_pp_bgh, l1_ll_Wg, l1_ll_bg, l1_ll_Wm, l1_ll_bm, l1_ll_Wi0, l1_ll_bi0, l1_ll_Wgi, l1_ll_Wgh, l1_ll_bgi, l1_ll_bgh, l1_lp_Wg, l1_lp_bg, l1_lp_Wm, l1_lp_bm, l1_lp_Wi0, l1_lp_bi0, l1_lp_Wgi, l1_lp_Wgh, l1_lp_bgi, l1_lp_bgh, l1_pl_Wg, l1_pl_bg, l1_pl_Wm, l1_pl_bm, l1_pl_Wi0, l1_pl_bi0, l1_pl_Wgi, l1_pl_Wgh, l1_pl_bgi, l1_pl_bgh, l2_pp_Ws, l2_pp_Wd, l2_pp_as, l2_pp_ad, l2_pp_Wgi, l2_pp_Wgh, l2_pp_bgi, l2_pp_bgh, l2_ll_Ws, l2_ll_Wd, l2_ll_as, l2_ll_ad, l2_ll_Wgi, l2_ll_Wgh, l2_ll_bgi, l2_ll_bgh, l2_lp_Ws, l2_lp_Wd, l2_lp_as, l2_lp_ad, l2_lp_Wgi, l2_lp_Wgh, l2_lp_bgi, l2_lp_bgh, l2_pl_Ws, l2_pl_Wd, l2_pl_as, l2_pl_ad, l2_pl_Wgi, l2_pl_Wgh, l2_pl_bgi, l2_pl_bgh, mol_pa_Ws, mol_pa_as, mol_pa_ad, mol_pa_Wgi, mol_pa_Wgh, mol_pa_bgi, mol_pa_bgh, mol_la_Ws, mol_la_as, mol_la_ad, mol_la_Wgi, mol_la_Wgh, mol_la_bgi, mol_la_bgh, lin_pa_W, lin_pa_b, lin_la_W, lin_la_b, mlp_W1, mlp_b1, mlp_W2, mlp_b2):
    ea = {"pp": _pad_cols(edge_attr_pp, 8), "ll": _pad_cols(edge_attr_ll, 8),
          "lp": _pad_cols(edge_attr_lp, 8), "pl": _pad_cols(edge_attr_pl, 8)}
    ei = {"pp": edge_index_pp, "ll": edge_index_ll,
          "lp": edge_index_lp, "pl": edge_index_pl}
    f = x_pa.shape[1]
    fp = 40
    xpa_pad = _pad_cols(x_pa, fp)
    xla_pad = _pad_cols(x_la, fp)

    def l1w(Wg, bg, Wm, bm, Wi0, bi0, Wgi, Wgh, bgi, bgh):
        return (_pad_rows(Wg[:f], fp), _pad_rows(Wg[f:2 * f], fp),
                _pad_rows(Wm[:f], fp), _pad_rows(Wg[2 * f:], 8),
                _pad_rows(Wm[f:], 8), bg, bm, Wi0, bi0, Wgi, Wgh, bgi, bgh)

    p1 = {
        "pp": l1w(l1_pp_Wg, l1_pp_bg, l1_pp_Wm, l1_pp_bm, l1_pp_Wi0, l1_pp_bi0, l1_pp_Wgi, l1_pp_Wgh, l1_pp_bgi, l1_pp_bgh),
        "ll": l1w(l1_ll_Wg, l1_ll_bg, l1_ll_Wm, l1_ll_bm, l1_ll_Wi0, l1_ll_bi0, l1_ll_Wgi, l1_ll_Wgh, l1_ll_bgi, l1_ll_bgh),
        "lp": l1w(l1_lp_Wg, l1_lp_bg, l1_lp_Wm, l1_lp_bm, l1_lp_Wi0, l1_lp_bi0, l1_lp_Wgi, l1_lp_Wgh, l1_lp_bgi, l1_lp_bgh),
        "pl": l1w(l1_pl_Wg, l1_pl_bg, l1_pl_Wm, l1_pl_bm, l1_pl_Wi0, l1_pl_bi0, l1_pl_Wgi, l1_pl_Wgh, l1_pl_bgi, l1_pl_bgh),
    }
    h_pa = _add_relu(
        _gate_conv(xpa_pad, xpa_pad, x_pa, ei["pp"], ea["pp"], *p1["pp"]),
        _gate_conv(xla_pad, xpa_pad, x_pa, ei["lp"], ea["lp"], *p1["lp"]))
    h_la = _add_relu(
        _gate_conv(xla_pad, xla_pad, x_la, ei["ll"], ea["ll"], *p1["ll"]),
        _gate_conv(xpa_pad, xla_pad, x_la, ei["pl"], ea["pl"], *p1["pl"]))

    g_pa = _add_relu(
        _gat_conv(h_pa, h_pa, ei["pp"], l2_pp_Ws, l2_pp_Wd, l2_pp_as, l2_pp_ad, l2_pp_Wgi, l2_pp_Wgh, l2_pp_bgi, l2_pp_bgh),
        _gat_conv(h_la, h_pa, ei["lp"], l2_lp_Ws, l2_lp_Wd, l2_lp_as, l2_lp_ad, l2_lp_Wgi, l2_lp_Wgh, l2_lp_bgi, l2_lp_bgh))
    g_la = _add_relu(
        _gat_conv(h_la, h_la, ei["ll"], l2_ll_Ws, l2_ll_Wd, l2_ll_as, l2_ll_ad, l2_ll_Wgi, l2_ll_Wgh, l2_ll_bgi, l2_ll_bgh),
        _gat_conv(h_pa, h_la, ei["pl"], l2_pl_Ws, l2_pl_Wd, l2_pl_as, l2_pl_ad, l2_pl_Wgi, l2_pl_Wgh, l2_pl_bgi, l2_pl_bgh))

    bpa = batch_pa.astype(jnp.int32)
    bla = batch_la.astype(jnp.int32)

    m_pa = _pool0(g_pa, bpa)
    m_la = _pool0(g_la, bla)

    hs_pa, ls_pa = _proj_attn(g_pa, mol_pa_Ws, mol_pa_as)
    hs_la, ls_la = _proj_attn(g_la, mol_la_Ws, mol_la_as)
    for _ in range(2):
        num, den = _molpool(bpa, ls_pa, hs_pa, m_pa, mol_pa_ad)
        m_pa = _gru(num, m_pa, mol_pa_Wgi, mol_pa_Wgh, mol_pa_bgi, mol_pa_bgh, den=den)
        num, den = _molpool(bla, ls_la, hs_la, m_la, mol_la_ad)
        m_la = _gru(num, m_la, mol_la_Wgi, mol_la_Wgh, mol_la_bgi, mol_la_bgh, den=den)

    return _head(m_pa, m_la, lin_pa_W, lin_pa_b, lin_la_W, lin_la_b,
                 mlp_W1, mlp_b1, mlp_W2, mlp_b2)
